# baseline (speedup 1.0000x reference)
import math

import jax
import jax.numpy as jnp
from jax import lax
from jax.experimental import pallas as pl
from jax.experimental.pallas import tpu as pltpu
from jax.experimental.pallas import tpu_sc as plsc

VOCAB = 100000
EMB = 128
B = 1024
L = 200
SCALE = math.sqrt(EMB)

_INFO = plsc.get_sparse_core_info()
NC, NS, LANES = _INFO.num_cores, _INFO.num_subcores, _INFO.num_lanes
NW = NC * NS

N_TOK = B * L
PER_W = N_TOK // NW
CHUNK = 128
N_CHUNKS = PER_W // CHUNK
NBUF = 5
PREF = 2


def _body(tokens_hbm, table_hbm, out_hbm, idx_v, bufs, sems_g, sems_s):
    wid = lax.axis_index("s") * NC + lax.axis_index("c")
    base = wid * PER_W
    pltpu.sync_copy(tokens_hbm.at[wid], idx_v)

    def start_gather(s, slot):
        pltpu.async_copy(
            table_hbm.at[idx_v.at[s]], bufs.at[slot], sems_g.at[slot]
        )

    def wait_gather(slot):
        pltpu.make_async_copy(
            table_hbm.at[pl.ds(0, CHUNK)], bufs.at[slot], sems_g.at[slot]
        ).wait()

    def start_scatter(s, slot):
        pltpu.async_copy(
            bufs.at[slot], out_hbm.at[pl.ds(base + s * CHUNK, CHUNK)],
            sems_s.at[slot],
        )

    def wait_scatter(slot):
        pltpu.make_async_copy(
            bufs.at[slot], out_hbm.at[pl.ds(base, CHUNK)], sems_s.at[slot]
        ).wait()

    def scale(slot):
        buf = bufs.at[slot]

        @pl.loop(0, CHUNK, unroll=2)
        def _row(r):
            for j in range(EMB // LANES):
                buf[r, pl.ds(j * LANES, LANES)] = (
                    buf[r, pl.ds(j * LANES, LANES)] * SCALE
                )

    def stage(s, slot):
        nxt = (slot + PREF) % NBUF
        pf = s + PREF < N_CHUNKS

        @pl.when(jnp.logical_and(pf, s + PREF >= NBUF))
        def _():
            wait_scatter(nxt)

        @pl.when(pf)
        def _():
            start_gather(s + PREF, nxt)

        wait_gather(slot)
        scale(slot)
        start_scatter(s, slot)

    for k in range(PREF):
        start_gather(k, k)

    @pl.loop(0, N_CHUNKS, step=NBUF)
    def _block(c):
        for b in range(NBUF):
            stage(c + b, b)

    for b in range(NBUF):
        wait_scatter(b)


@jax.jit
def _embed(tokens_grouped, table):
    kfn = pl.kernel(
        _body,
        out_type=jax.ShapeDtypeStruct((N_TOK, EMB), jnp.float32),
        mesh=plsc.VectorSubcoreMesh(core_axis_name="c", subcore_axis_name="s"),
        scratch_types=[
            pltpu.VMEM((N_CHUNKS, CHUNK), jnp.int32),
            pltpu.VMEM((NBUF, CHUNK, EMB), jnp.float32),
            pltpu.SemaphoreType.DMA((NBUF,)),
            pltpu.SemaphoreType.DMA((NBUF,)),
        ],
    )
    return kfn(tokens_grouped, table)


def kernel(tokens, table):
    tokens_grouped = tokens.reshape(NW, N_CHUNKS, CHUNK).astype(jnp.int32)
    out = _embed(tokens_grouped, table)
    return out.reshape(B, L, EMB)


# speedup vs baseline: 1.0076x; 1.0076x over previous
import math

import jax
import jax.numpy as jnp
from jax import lax
from jax.experimental import pallas as pl
from jax.experimental.pallas import tpu as pltpu
from jax.experimental.pallas import tpu_sc as plsc

VOCAB = 100000
EMB = 128
B = 1024
L = 200
SCALE = math.sqrt(EMB)

_INFO = plsc.get_sparse_core_info()
NC, NS, LANES = _INFO.num_cores, _INFO.num_subcores, _INFO.num_lanes
NW = NC * NS

N_TOK = B * L
PER_W = N_TOK // NW
CHUNK = 128
N_CHUNKS = PER_W // CHUNK
NBUF = 5
PREF = 2


def _body(tokens_hbm, table_hbm, out_hbm, idx_v, bufs, sems_g, sems_s):
    wid = lax.axis_index("s") * NC + lax.axis_index("c")
    base = wid * PER_W
    pltpu.sync_copy(tokens_hbm.at[wid], idx_v)

    def start_gather(s, slot):
        pltpu.async_copy(
            table_hbm.at[idx_v.at[s]], bufs.at[slot], sems_g.at[slot]
        )

    def wait_gather(slot):
        pltpu.make_async_copy(
            table_hbm.at[pl.ds(0, CHUNK)], bufs.at[slot], sems_g.at[slot]
        ).wait()

    def start_scatter(s, slot):
        pltpu.async_copy(
            bufs.at[slot], out_hbm.at[pl.ds(base + s * CHUNK, CHUNK)],
            sems_s.at[slot],
        )

    def wait_scatter(slot):
        pltpu.make_async_copy(
            bufs.at[slot], out_hbm.at[pl.ds(base, CHUNK)], sems_s.at[slot]
        ).wait()

    def scale(slot):
        buf = bufs.at[slot]

        @pl.loop(0, CHUNK, unroll=4)
        def _row(r):
            for j in range(EMB // LANES):
                buf[r, pl.ds(j * LANES, LANES)] = (
                    buf[r, pl.ds(j * LANES, LANES)] * SCALE
                )

    def stage(s, slot):
        nxt = (slot + PREF) % NBUF
        pf = s + PREF < N_CHUNKS

        @pl.when(jnp.logical_and(pf, s + PREF >= NBUF))
        def _():
            wait_scatter(nxt)

        @pl.when(pf)
        def _():
            start_gather(s + PREF, nxt)

        wait_gather(slot)
        scale(slot)
        start_scatter(s, slot)

    for k in range(PREF):
        start_gather(k, k)

    @pl.loop(0, N_CHUNKS, step=NBUF)
    def _block(c):
        for b in range(NBUF):
            stage(c + b, b)

    for b in range(NBUF):
        wait_scatter(b)


@jax.jit
def _embed(tokens_grouped, table):
    kfn = pl.kernel(
        _body,
        out_type=jax.ShapeDtypeStruct((N_TOK, EMB), jnp.float32),
        mesh=plsc.VectorSubcoreMesh(core_axis_name="c", subcore_axis_name="s"),
        scratch_types=[
            pltpu.VMEM((N_CHUNKS, CHUNK), jnp.int32),
            pltpu.VMEM((NBUF, CHUNK, EMB), jnp.float32),
            pltpu.SemaphoreType.DMA((NBUF,)),
            pltpu.SemaphoreType.DMA((NBUF,)),
        ],
    )
    return kfn(tokens_grouped, table)


def kernel(tokens, table):
    tokens_grouped = tokens.reshape(NW, N_CHUNKS, CHUNK).astype(jnp.int32)
    out = _embed(tokens_grouped, table)
    return out.reshape(B, L, EMB)
